# Initial kernel scaffold; baseline (speedup 1.0000x reference)
import jax
import jax.numpy as jnp
from jax.experimental import pallas as pl


def kernel(read_count_bucket, write_count_bucket, fault_count_bucket, cow_count_bucket, recency_bucket, volatility_features, pressure_features, count_table, recency_table, Wp1, bp1, Wp2, bp2, Wv1, bv1, Wv2, bv2, Wg, bg, Wt, bt, gamma, beta):
    raise NotImplementedError("write your pallas kernel here")


# speedup vs baseline: 1.0228x; 1.0228x over previous
import jax
import jax.numpy as jnp
from jax import lax
from jax.experimental import pallas as pl
from jax.experimental.pallas import tpu as pltpu
from jax.experimental.pallas import tpu_sc as plsc

_B = 16384
_D = 128
_E = 8
_NF = 5
_NW = 32
_CHUNK = 128

_CNT_ROWS = 4 * _B // _NW
_REC_ROWS = _B // _NW
_CNT_CH = _CNT_ROWS // _CHUNK
_REC_CH = _REC_ROWS // _CHUNK
_N_CH = _CNT_CH + _REC_CH


def _sc_gather(count_hbm, recency_hbm, idx_hbm, out_hbm, idx_v, rows_v, sem):
    wid = lax.axis_index("s") * 2 + lax.axis_index("c")
    cbase = wid * _CNT_CH
    rbase = (4 * _B // _CHUNK) + wid * _REC_CH
    pltpu.sync_copy(idx_hbm.at[pl.ds(cbase, _CNT_CH)], idx_v.at[pl.ds(0, _CNT_CH)])
    pltpu.sync_copy(idx_hbm.at[pl.ds(rbase, _REC_CH)], idx_v.at[pl.ds(_CNT_CH, _REC_CH)])
    copies = []
    for j in range(_CNT_CH):
        copies.append(
            pltpu.async_copy(count_hbm.at[idx_v.at[j]], rows_v.at[j], sem))
    for j in range(_REC_CH):
        copies.append(
            pltpu.async_copy(recency_hbm.at[idx_v.at[_CNT_CH + j]],
                             rows_v.at[_CNT_CH + j], sem))
    for cp in copies:
        cp.wait()
    pltpu.sync_copy(rows_v.at[pl.ds(0, _CNT_CH)],
                    out_hbm.at[pl.ds(cbase, _CNT_CH)])
    pltpu.sync_copy(rows_v.at[pl.ds(_CNT_CH, _REC_CH)],
                    out_hbm.at[pl.ds(rbase, _REC_CH)])


def _gather_embeddings(count_pad, recency_pad, idx_all):
    nrows = 5 * _B // _CHUNK
    mesh = plsc.VectorSubcoreMesh(core_axis_name="c", subcore_axis_name="s")
    fn = pl.kernel(
        _sc_gather,
        out_type=jax.ShapeDtypeStruct((nrows, _CHUNK, _E), jnp.float32),
        mesh=mesh,
        scratch_types=[
            pltpu.VMEM((_N_CH, _CHUNK), jnp.int32),
            pltpu.VMEM((_N_CH, _CHUNK, _E), jnp.float32),
            pltpu.SemaphoreType.DMA,
        ],
        compiler_params=pltpu.CompilerParams(use_tc_tiling_on_sc=False),
    )
    return fn(count_pad, recency_pad, idx_all)


def _gelu(x):
    return 0.5 * x * (1.0 + lax.erf(x * 0.7071067811865476))


def _dense_body(e_ref, vol_ref, press_ref,
                wv1_ref, bv1_ref, wv2_ref, bv2_ref,
                wp1_ref, bp1_ref, wp2_ref, bp2_ref,
                wg_ref, bg_ref, wt_ref, bt_ref,
                gamma_ref, beta_ref, out_ref):
    vol = vol_ref[...]
    press = press_ref[...]
    hv = _gelu(jnp.dot(vol, wv1_ref[...], preferred_element_type=jnp.float32)
               + bv1_ref[...])
    pv = jnp.dot(hv, wv2_ref[...], preferred_element_type=jnp.float32) + bv2_ref[...]
    hp = _gelu(jnp.dot(press, wp1_ref[...], preferred_element_type=jnp.float32)
               + bp1_ref[...])
    pp = jnp.dot(hp, wp2_ref[...], preferred_element_type=jnp.float32) + bp2_ref[...]
    e = e_ref[...]
    combined = jnp.concatenate(
        [e[0][:, :5], e[1][:, :5], e[2][:, :5], e[3][:, :5], e[4][:, :5],
         pv, pp], axis=-1)
    zg = jnp.dot(combined, wg_ref[...], preferred_element_type=jnp.float32) + bg_ref[...]
    zt = jnp.dot(combined, wt_ref[...], preferred_element_type=jnp.float32) + bt_ref[...]
    z = jax.nn.sigmoid(zg * 1.2) * zt
    mu = jnp.mean(z, axis=-1, keepdims=True)
    zc = z - mu
    var = jnp.mean(zc * zc, axis=-1, keepdims=True)
    out_ref[...] = zc * lax.rsqrt(var + 1e-5) * gamma_ref[...] + beta_ref[...]


def kernel(read_count_bucket, write_count_bucket, fault_count_bucket,
           cow_count_bucket, recency_bucket, volatility_features,
           pressure_features, count_table, recency_table,
           Wp1, bp1, Wp2, bp2, Wv1, bv1, Wv2, bv2,
           Wg, bg, Wt, bt, gamma, beta):
    idx_all = jnp.concatenate(
        [read_count_bucket, write_count_bucket, fault_count_bucket,
         cow_count_bucket, recency_bucket]).reshape(5 * _B // _CHUNK, _CHUNK)
    count_pad = jnp.pad(count_table, ((0, 0), (0, 3)))
    recency_pad = jnp.pad(recency_table, ((0, 0), (0, 3)))
    e_all = _gather_embeddings(count_pad, recency_pad, idx_all)
    e_all = e_all.reshape(_NF, _B, _E)

    bB = 2048
    grid = _B // bB
    out = pl.pallas_call(
        _dense_body,
        grid=(grid,),
        in_specs=[
            pl.BlockSpec((_NF, bB, _E), lambda i: (0, i, 0)),
            pl.BlockSpec((bB, 4), lambda i: (i, 0)),
            pl.BlockSpec((bB, 12), lambda i: (i, 0)),
            pl.BlockSpec((4, 8), lambda i: (0, 0)),
            pl.BlockSpec((8,), lambda i: (0,)),
            pl.BlockSpec((8, 6), lambda i: (0, 0)),
            pl.BlockSpec((6,), lambda i: (0,)),
            pl.BlockSpec((12, 24), lambda i: (0, 0)),
            pl.BlockSpec((24,), lambda i: (0,)),
            pl.BlockSpec((24, 20), lambda i: (0, 0)),
            pl.BlockSpec((20,), lambda i: (0,)),
            pl.BlockSpec((51, _D), lambda i: (0, 0)),
            pl.BlockSpec((_D,), lambda i: (0,)),
            pl.BlockSpec((51, _D), lambda i: (0, 0)),
            pl.BlockSpec((_D,), lambda i: (0,)),
            pl.BlockSpec((_D,), lambda i: (0,)),
            pl.BlockSpec((_D,), lambda i: (0,)),
        ],
        out_specs=pl.BlockSpec((bB, _D), lambda i: (i, 0)),
        out_shape=jax.ShapeDtypeStruct((_B, _D), jnp.float32),
    )(e_all, volatility_features, pressure_features,
      Wv1, bv1, Wv2, bv2, Wp1, bp1, Wp2, bp2,
      Wg, bg, Wt, bt, gamma, beta)
    return out


# speedup vs baseline: 1.7782x; 1.7386x over previous
import jax
import jax.numpy as jnp
from jax import lax
from jax.experimental import pallas as pl
from jax.experimental.pallas import tpu as pltpu
from jax.experimental.pallas import tpu_sc as plsc

_B = 16384
_V = 1000000
_D = 128
_NC = 25
_NW = 32
_BT = _B // _NW
_CH = _BT // 128


def _sc_gather(src_hbm, idx_hbm, out_hbm, idx_v, rows_v, sem):
    wid = lax.axis_index("s") * 2 + lax.axis_index("c")
    pltpu.sync_copy(idx_hbm.at[:, pl.ds(wid * _BT, _BT)], idx_v)
    copies = []
    for r in range(_NC):
        for c in range(_CH):
            copies.append(pltpu.async_copy(
                src_hbm.at[idx_v.at[r, pl.ds(c * 128, 128)]],
                rows_v.at[r, pl.ds(c * 128, 128)], sem))
    for cp in copies:
        cp.wait()
    pltpu.sync_copy(rows_v, out_hbm.at[:, pl.ds(wid * _BT, _BT)])


def _gather_embeddings(src_flat, idx25):
    mesh = plsc.VectorSubcoreMesh(core_axis_name="c", subcore_axis_name="s")
    fn = pl.kernel(
        _sc_gather,
        out_type=jax.ShapeDtypeStruct((_NC, _B), jnp.float32),
        mesh=mesh,
        scratch_types=[
            pltpu.VMEM((_NC, _BT), jnp.int32),
            pltpu.VMEM((_NC, _BT), jnp.float32),
            pltpu.SemaphoreType.DMA,
        ],
        compiler_params=pltpu.CompilerParams(use_tc_tiling_on_sc=False),
    )
    return fn(src_flat, idx25)


def _gelu(x):
    return 0.5 * x * (1.0 + lax.erf(x * 0.7071067811865476))


def _dot0(w_ref, x):
    return lax.dot_general(w_ref[...], x, (((0,), (0,)), ((), ())),
                           preferred_element_type=jnp.float32)


def _dense_body(e_ref, vol_ref, press_ref,
                wv1_ref, bv1_ref, wv2_ref, bv2_ref,
                wp1_ref, bp1_ref, wp2_ref, bp2_ref,
                wg_ref, bg_ref, wt_ref, bt_ref,
                gamma_ref, beta_ref, out_ref):
    hv = _gelu(_dot0(wv1_ref, vol_ref[...]) + bv1_ref[...][:, None])
    pv = _dot0(wv2_ref, hv) + bv2_ref[...][:, None]
    hp = _gelu(_dot0(wp1_ref, press_ref[...]) + bp1_ref[...][:, None])
    pp = _dot0(wp2_ref, hp) + bp2_ref[...][:, None]
    combined = jnp.concatenate([e_ref[...], pv, pp], axis=0)
    zg = _dot0(wg_ref, combined) + bg_ref[...][:, None]
    zt = _dot0(wt_ref, combined) + bt_ref[...][:, None]
    z = jax.nn.sigmoid(zg * 1.2) * zt
    mu = jnp.mean(z, axis=0, keepdims=True)
    zc = z - mu
    var = jnp.mean(zc * zc, axis=0, keepdims=True)
    y = zc * lax.rsqrt(var + 1e-5) * gamma_ref[...][:, None] + beta_ref[...][:, None]
    out_ref[...] = y.T


def kernel(read_count_bucket, write_count_bucket, fault_count_bucket,
           cow_count_bucket, recency_bucket, volatility_features,
           pressure_features, count_table, recency_table,
           Wp1, bp1, Wp2, bp2, Wv1, bv1, Wv2, bv2,
           Wg, bg, Wt, bt, gamma, beta):
    src_flat = jnp.concatenate(
        [count_table.T.reshape(-1), recency_table.T.reshape(-1)])
    base = jnp.stack([read_count_bucket, write_count_bucket,
                      fault_count_bucket, cow_count_bucket,
                      recency_bucket])
    planes = (jnp.arange(5, dtype=jnp.int32) * _V)[None, :, None]
    off_f = jnp.array([0, 0, 0, 0, 5 * _V], jnp.int32)[:, None, None]
    idx25 = (base[:, None, :] + planes + off_f).reshape(_NC, _B)
    e25 = _gather_embeddings(src_flat, idx25)

    bB = 2048
    grid = _B // bB
    out = pl.pallas_call(
        _dense_body,
        grid=(grid,),
        in_specs=[
            pl.BlockSpec((_NC, bB), lambda i: (0, i)),
            pl.BlockSpec((4, bB), lambda i: (0, i)),
            pl.BlockSpec((12, bB), lambda i: (0, i)),
            pl.BlockSpec((4, 8), lambda i: (0, 0)),
            pl.BlockSpec((8,), lambda i: (0,)),
            pl.BlockSpec((8, 6), lambda i: (0, 0)),
            pl.BlockSpec((6,), lambda i: (0,)),
            pl.BlockSpec((12, 24), lambda i: (0, 0)),
            pl.BlockSpec((24,), lambda i: (0,)),
            pl.BlockSpec((24, 20), lambda i: (0, 0)),
            pl.BlockSpec((20,), lambda i: (0,)),
            pl.BlockSpec((51, _D), lambda i: (0, 0)),
            pl.BlockSpec((_D,), lambda i: (0,)),
            pl.BlockSpec((51, _D), lambda i: (0, 0)),
            pl.BlockSpec((_D,), lambda i: (0,)),
            pl.BlockSpec((_D,), lambda i: (0,)),
            pl.BlockSpec((_D,), lambda i: (0,)),
        ],
        out_specs=pl.BlockSpec((bB, _D), lambda i: (i, 0)),
        out_shape=jax.ShapeDtypeStruct((_B, _D), jnp.float32),
    )(e25, volatility_features.T, pressure_features.T,
      Wv1, bv1, Wv2, bv2, Wp1, bp1, Wp2, bp2,
      Wg, bg, Wt, bt, gamma, beta)
    return out


# speedup vs baseline: 17.6667x; 9.9350x over previous
import jax
import jax.numpy as jnp
from jax import lax
from jax.experimental import pallas as pl
from jax.experimental.pallas import tpu as pltpu
from jax.experimental.pallas import tpu_sc as plsc

_B = 16384
_V = 1000000
_D = 128
_NC = 25
_NW = 32
_BT = _B // _NW
_CH = _BT // 128
_BLK = 1 << 17
_CPP = -(-_V // _BLK)


def _detile_body(ct_ref, rt_ref, *out_refs):
    for j in range(5):
        out_refs[j][...] = ct_ref[0, j, :]
        out_refs[5 + j][...] = rt_ref[0, j, :]


def _detile(count_table, recency_table):
    ct = count_table.T.reshape(1, 5, _V)
    rt = recency_table.T.reshape(1, 5, _V)
    return pl.pallas_call(
        _detile_body,
        grid=(_CPP,),
        in_specs=[pl.BlockSpec((1, 5, _BLK), lambda c: (0, 0, c)),
                  pl.BlockSpec((1, 5, _BLK), lambda c: (0, 0, c))],
        out_specs=[pl.BlockSpec((_BLK,), lambda c: (c,))] * 10,
        out_shape=[jax.ShapeDtypeStruct((_V,), jnp.float32)] * 10,
    )(ct, rt)


def _sc_gather(*refs):
    srcs = refs[:10]
    idx_hbm, out_hbm, idx_v, rows_v, sem = refs[10:]
    wid = lax.axis_index("s") * 2 + lax.axis_index("c")
    pltpu.sync_copy(idx_hbm.at[:, pl.ds(wid * _BT, _BT)], idx_v)
    copies = []
    for r in range(_NC):
        f, j = divmod(r, 5)
        src = srcs[j] if f < 4 else srcs[5 + j]
        for c in range(_CH):
            copies.append(pltpu.async_copy(
                src.at[idx_v.at[f, pl.ds(c * 128, 128)]],
                rows_v.at[r, pl.ds(c * 128, 128)], sem))
    for cp in copies:
        cp.wait()
    pltpu.sync_copy(rows_v, out_hbm.at[:, pl.ds(wid * _BT, _BT)])


def _gather_embeddings(planes, base_idx):
    mesh = plsc.VectorSubcoreMesh(core_axis_name="c", subcore_axis_name="s")
    fn = pl.kernel(
        _sc_gather,
        out_type=jax.ShapeDtypeStruct((_NC, _B), jnp.float32),
        mesh=mesh,
        scratch_types=[
            pltpu.VMEM((5, _BT), jnp.int32),
            pltpu.VMEM((_NC, _BT), jnp.float32),
            pltpu.SemaphoreType.DMA,
        ],
        compiler_params=pltpu.CompilerParams(use_tc_tiling_on_sc=False),
    )
    return fn(*planes, base_idx)


def _gelu(x):
    return 0.5 * x * (1.0 + lax.erf(x * 0.7071067811865476))


def _dot0(w_ref, x):
    return lax.dot_general(w_ref[...], x, (((0,), (0,)), ((), ())),
                           preferred_element_type=jnp.float32)


def _dense_body(e_ref, vol_ref, press_ref,
                wv1_ref, bv1_ref, wv2_ref, bv2_ref,
                wp1_ref, bp1_ref, wp2_ref, bp2_ref,
                wg_ref, bg_ref, wt_ref, bt_ref,
                gamma_ref, beta_ref, out_ref):
    hv = _gelu(_dot0(wv1_ref, vol_ref[...]) + bv1_ref[...][:, None])
    pv = _dot0(wv2_ref, hv) + bv2_ref[...][:, None]
    hp = _gelu(_dot0(wp1_ref, press_ref[...]) + bp1_ref[...][:, None])
    pp = _dot0(wp2_ref, hp) + bp2_ref[...][:, None]
    combined = jnp.concatenate([e_ref[...], pv, pp], axis=0)
    zg = _dot0(wg_ref, combined) + bg_ref[...][:, None]
    zt = _dot0(wt_ref, combined) + bt_ref[...][:, None]
    z = jax.nn.sigmoid(zg * 1.2) * zt
    mu = jnp.mean(z, axis=0, keepdims=True)
    zc = z - mu
    var = jnp.mean(zc * zc, axis=0, keepdims=True)
    y = zc * lax.rsqrt(var + 1e-5) * gamma_ref[...][:, None] + beta_ref[...][:, None]
    out_ref[...] = y.T


def kernel(read_count_bucket, write_count_bucket, fault_count_bucket,
           cow_count_bucket, recency_bucket, volatility_features,
           pressure_features, count_table, recency_table,
           Wp1, bp1, Wp2, bp2, Wv1, bv1, Wv2, bv2,
           Wg, bg, Wt, bt, gamma, beta):
    planes = _detile(count_table, recency_table)
    base = jnp.stack([read_count_bucket, write_count_bucket,
                      fault_count_bucket, cow_count_bucket,
                      recency_bucket])
    e25 = _gather_embeddings(planes, base)

    bB = 2048
    grid = _B // bB
    out = pl.pallas_call(
        _dense_body,
        grid=(grid,),
        in_specs=[
            pl.BlockSpec((_NC, bB), lambda i: (0, i)),
            pl.BlockSpec((4, bB), lambda i: (0, i)),
            pl.BlockSpec((12, bB), lambda i: (0, i)),
            pl.BlockSpec((4, 8), lambda i: (0, 0)),
            pl.BlockSpec((8,), lambda i: (0,)),
            pl.BlockSpec((8, 6), lambda i: (0, 0)),
            pl.BlockSpec((6,), lambda i: (0,)),
            pl.BlockSpec((12, 24), lambda i: (0, 0)),
            pl.BlockSpec((24,), lambda i: (0,)),
            pl.BlockSpec((24, 20), lambda i: (0, 0)),
            pl.BlockSpec((20,), lambda i: (0,)),
            pl.BlockSpec((51, _D), lambda i: (0, 0)),
            pl.BlockSpec((_D,), lambda i: (0,)),
            pl.BlockSpec((51, _D), lambda i: (0, 0)),
            pl.BlockSpec((_D,), lambda i: (0,)),
            pl.BlockSpec((_D,), lambda i: (0,)),
            pl.BlockSpec((_D,), lambda i: (0,)),
        ],
        out_specs=pl.BlockSpec((bB, _D), lambda i: (i, 0)),
        out_shape=jax.ShapeDtypeStruct((_B, _D), jnp.float32),
    )(e25, volatility_features.T, pressure_features.T,
      Wv1, bv1, Wv2, bv2, Wp1, bp1, Wp2, bp2,
      Wg, bg, Wt, bt, gamma, beta)
    return out


# speedup vs baseline: 17.8844x; 1.0123x over previous
import jax
import jax.numpy as jnp
from jax import lax
from jax.experimental import pallas as pl
from jax.experimental.pallas import tpu as pltpu
from jax.experimental.pallas import tpu_sc as plsc

_B = 16384
_V = 1000000
_D = 128
_NC = 25
_NW = 32
_BT = _B // _NW
_CH = _BT // 128
_BLK = 1 << 17
_S = 1 << 20
_CPP = _S // _BLK


def _detile_body(ct_ref, rt_ref, *out_refs):
    for j in range(5):
        out_refs[j][...] = ct_ref[0, j, :]
        out_refs[5 + j][...] = rt_ref[0, j, :]


def _detile(count_table, recency_table):
    ct = count_table.T.reshape(1, 5, _V)
    rt = recency_table.T.reshape(1, 5, _V)
    return pl.pallas_call(
        _detile_body,
        grid=(_CPP,),
        in_specs=[pl.BlockSpec((1, 5, _BLK), lambda c: (0, 0, c)),
                  pl.BlockSpec((1, 5, _BLK), lambda c: (0, 0, c))],
        out_specs=[pl.BlockSpec((_BLK,), lambda c: (c,))] * 10,
        out_shape=[jax.ShapeDtypeStruct((_S,), jnp.float32)] * 10,
    )(ct, rt)


def _sc_gather(*refs):
    srcs = refs[:10]
    idx_hbm, out_hbm, idx_v, rows_v, sem = refs[10:]
    wid = lax.axis_index("s") * 2 + lax.axis_index("c")
    pltpu.sync_copy(idx_hbm.at[:, pl.ds(wid * _BT, _BT)], idx_v)
    copies = []
    for r in range(_NC):
        f, j = divmod(r, 5)
        src = srcs[j] if f < 4 else srcs[5 + j]
        copies.append(pltpu.async_copy(
            src.at[idx_v.at[f]], rows_v.at[r], sem))
    for cp in copies:
        cp.wait()
    pltpu.sync_copy(rows_v, out_hbm.at[:, pl.ds(wid * _BT, _BT)])


def _gather_embeddings(planes, base_idx):
    mesh = plsc.VectorSubcoreMesh(core_axis_name="c", subcore_axis_name="s")
    fn = pl.kernel(
        _sc_gather,
        out_type=jax.ShapeDtypeStruct((_NC, _B), jnp.float32),
        mesh=mesh,
        scratch_types=[
            pltpu.VMEM((5, _BT), jnp.int32),
            pltpu.VMEM((_NC, _BT), jnp.float32),
            pltpu.SemaphoreType.DMA,
        ],
        compiler_params=pltpu.CompilerParams(use_tc_tiling_on_sc=False),
    )
    return fn(*planes, base_idx)


def _gelu(x):
    return 0.5 * x * (1.0 + lax.erf(x * 0.7071067811865476))


def _dot0(w_ref, x):
    return lax.dot_general(w_ref[...], x, (((0,), (0,)), ((), ())),
                           preferred_element_type=jnp.float32)


def _dense_body(e_ref, vol_ref, press_ref,
                wv1_ref, bv1_ref, wv2_ref, bv2_ref,
                wp1_ref, bp1_ref, wp2_ref, bp2_ref,
                wg_ref, bg_ref, wt_ref, bt_ref,
                gamma_ref, beta_ref, out_ref):
    hv = _gelu(_dot0(wv1_ref, vol_ref[...]) + bv1_ref[...][:, None])
    pv = _dot0(wv2_ref, hv) + bv2_ref[...][:, None]
    hp = _gelu(_dot0(wp1_ref, press_ref[...]) + bp1_ref[...][:, None])
    pp = _dot0(wp2_ref, hp) + bp2_ref[...][:, None]
    combined = jnp.concatenate([e_ref[...], pv, pp], axis=0)
    zg = _dot0(wg_ref, combined) + bg_ref[...][:, None]
    zt = _dot0(wt_ref, combined) + bt_ref[...][:, None]
    z = jax.nn.sigmoid(zg * 1.2) * zt
    mu = jnp.mean(z, axis=0, keepdims=True)
    zc = z - mu
    var = jnp.mean(zc * zc, axis=0, keepdims=True)
    y = zc * lax.rsqrt(var + 1e-5) * gamma_ref[...][:, None] + beta_ref[...][:, None]
    out_ref[...] = y.T


def kernel(read_count_bucket, write_count_bucket, fault_count_bucket,
           cow_count_bucket, recency_bucket, volatility_features,
           pressure_features, count_table, recency_table,
           Wp1, bp1, Wp2, bp2, Wv1, bv1, Wv2, bv2,
           Wg, bg, Wt, bt, gamma, beta):
    planes = _detile(count_table, recency_table)
    base = jnp.stack([read_count_bucket, write_count_bucket,
                      fault_count_bucket, cow_count_bucket,
                      recency_bucket])
    e25 = _gather_embeddings(planes, base)

    bB = 2048
    grid = _B // bB
    out = pl.pallas_call(
        _dense_body,
        grid=(grid,),
        in_specs=[
            pl.BlockSpec((_NC, bB), lambda i: (0, i)),
            pl.BlockSpec((4, bB), lambda i: (0, i)),
            pl.BlockSpec((12, bB), lambda i: (0, i)),
            pl.BlockSpec((4, 8), lambda i: (0, 0)),
            pl.BlockSpec((8,), lambda i: (0,)),
            pl.BlockSpec((8, 6), lambda i: (0, 0)),
            pl.BlockSpec((6,), lambda i: (0,)),
            pl.BlockSpec((12, 24), lambda i: (0, 0)),
            pl.BlockSpec((24,), lambda i: (0,)),
            pl.BlockSpec((24, 20), lambda i: (0, 0)),
            pl.BlockSpec((20,), lambda i: (0,)),
            pl.BlockSpec((51, _D), lambda i: (0, 0)),
            pl.BlockSpec((_D,), lambda i: (0,)),
            pl.BlockSpec((51, _D), lambda i: (0, 0)),
            pl.BlockSpec((_D,), lambda i: (0,)),
            pl.BlockSpec((_D,), lambda i: (0,)),
            pl.BlockSpec((_D,), lambda i: (0,)),
        ],
        out_specs=pl.BlockSpec((bB, _D), lambda i: (i, 0)),
        out_shape=jax.ShapeDtypeStruct((_B, _D), jnp.float32),
    )(e25, volatility_features.T, pressure_features.T,
      Wv1, bv1, Wv2, bv2, Wp1, bp1, Wp2, bp2,
      Wg, bg, Wt, bt, gamma, beta)
    return out
